# baseline (speedup 1.0000x reference)
import functools
import numpy as np
import jax
import jax.numpy as jnp
from jax import lax
from jax.experimental import pallas as pl
from jax.experimental.pallas import tpu as pltpu
from jax.experimental.pallas import tpu_sc as plsc

B = 4096
S = 20
EMB = 36
F = 48
OUT = 256
NTOT = 2 * B * S
PER_W = NTOT // 32
CHUNK = 128


def _pad48(x):
    z = jnp.zeros((x.shape[0], F - EMB), jnp.float32)
    return jnp.concatenate([x, z], axis=1)


def _pk_body(pc_emb, addr_emb, enc_W, enc_b, p_all, p_ad):
    dn = (((1,), (1,)), ((), ()))
    for st, emb in enumerate((pc_emb[...], addr_emb[...])):
        for i in range(4):
            w_i = enc_W[:, EMB * i:EMB * (i + 1)]
            p = lax.dot_general(emb[i], w_i, dn,
                                preferred_element_type=jnp.float32)
            if st == 1:
                p_ad[i] = p
            if i == 0:
                p = p + enc_b[...]
            p_all[st * 4 + i] = p


def _t01_body(p_all, t01):
    st = pl.program_id(0)
    g = pl.program_id(1)
    a0 = _pad48(p_all[st * 4, pl.ds(g * 32, 32), :])
    a1 = _pad48(p_all[st * 4 + 1])
    t01[...] = jnp.reshape(a0[:, None, :] + a1[None, :, :], (8192, F))


def _t23_body(p_all, t23):
    st = pl.program_id(0)
    g = pl.program_id(1)
    a2 = _pad48(p_all[st * 4 + 2, pl.ds(g * 32, 32), :])
    a3 = _pad48(p_all[st * 4 + 3])
    t23[...] = jnp.reshape(a2[:, None, :] + a3[None, :, :], (8192, F))


def _sc_body(vals_hbm, t01_hbm, t23_hbm, o01_hbm, o23_hbm,
             vals_v, idx01_v, idx23_v, row_v, g01_v, g23_v, sem1, sem2):
    wid = lax.axis_index("s") * 2 + lax.axis_index("c")
    st = wid // 16
    base = wid * PER_W
    nloc0 = (wid % 16) * PER_W
    off01 = st * 32768
    off23 = st * 65536
    rowoff = st * B
    pltpu.sync_copy(vals_hbm.at[pl.ds(base, PER_W)], vals_v)

    @pl.loop(0, PER_W, step=CHUNK)
    def _chunk(co):
        @pl.loop(0, CHUNK, step=16)
        def _idx(k):
            v = vals_v[pl.ds(co + k, 16)]
            idx01_v[pl.ds(k, 16)] = (v >> 16) + off01
            idx23_v[pl.ds(k, 16)] = (v & 0xFFFF) + off23
            n = (nloc0 + co + k) + lax.iota(jnp.int32, 16)
            row_v[pl.ds(k, 16)] = (n % S) * (2 * B) + rowoff + (n // S)

        c1 = pltpu.async_copy(t01_hbm.at[idx01_v], g01_v, sem1)
        c2 = pltpu.async_copy(t23_hbm.at[idx23_v], g23_v, sem2)
        c1.wait()
        c2.wait()
        c3 = pltpu.async_copy(g01_v, o01_hbm.at[row_v], sem1)
        c4 = pltpu.async_copy(g23_v, o23_hbm.at[row_v], sem2)
        c3.wait()
        c4.wait()


def _dense_body(bb, o01pc, o23pc, o01ad, o23ad, w1pc, w1ad, b1w, w2m, b2w,
                w3m, b3w, decm, dec_b, p_ad, enc_b, rf1m, rf1b, rf2m, rf2b,
                probs0, probs1, probs2, probs3, log0, log1, log2, log3, fr):
    def widen(a, c):
        x = jax.nn.sigmoid(a[...] + c[...])
        pieces = [x[s] for s in range(S)]
        wide = jnp.concatenate(pieces, axis=1)
        d = pieces[0]
        for s in range(1, S):
            d = d + pieces[s]
        return wide, d * (1.0 / S)

    xp, dist_pc = widen(o01pc, o23pc)
    xa, dist_ad = widen(o01ad, o23ad)

    h1 = jnp.dot(xp, w1pc[...], preferred_element_type=jnp.float32) \
        + jnp.dot(xa, w1ad[...], preferred_element_type=jnp.float32) \
        + b1w[...]
    h2 = jnp.dot(h1, w2m[...], preferred_element_type=jnp.float32) + b2w[...]
    h3 = jnp.dot(h2, w3m[...], preferred_element_type=jnp.float32) + b3w[...]
    h24 = jnp.maximum(h3[:, 0:24], h3[:, 24:48])

    probs_refs = (probs0, probs1, probs2, probs3)
    log_refs = (log0, log1, log2, log3)
    fe_pre = None
    for i in range(4):
        logits = jnp.dot(h24, decm[i], preferred_element_type=jnp.float32) \
            + dec_b[i][None, :]
        log_refs[i][...] = logits
        ls = logits / 0.01
        m = jnp.max(ls, axis=1, keepdims=True)
        e = jnp.exp(ls - m)
        p = e / jnp.sum(e, axis=1, keepdims=True)
        probs_refs[i][...] = p
        contrib = jnp.dot(p, p_ad[i], preferred_element_type=jnp.float32)
        fe_pre = contrib if fe_pre is None else fe_pre + contrib

    fe = jax.nn.sigmoid(fe_pre + enc_b[...])
    feat = jnp.concatenate([fe, dist_pc[:, :EMB], dist_ad[:, :EMB]], axis=1)
    r1 = jnp.maximum(
        jnp.dot(feat, rf1m[...], preferred_element_type=jnp.float32)
        + rf1b[...], 0.0)
    fr[...] = jnp.dot(r1, rf2m[...], preferred_element_type=jnp.float32) \
        + rf2b[...]


def _conv_mats(conv1_w, conv2_w, conv3_w):
    r_pc, c_pc, v_pc, r_ad, c_ad, v_ad = [], [], [], [], [], []
    for o in range(20):
        for s in range(20):
            for j in range(35):
                for k in range(3):
                    f = 2 * j + k
                    vi = (o * 20 + s) * 3 + k
                    col = o * 35 + j
                    if f < 36:
                        r_pc.append(s * F + f); c_pc.append(col); v_pc.append(vi)
                    else:
                        r_ad.append(s * F + (f - 36)); c_ad.append(col); v_ad.append(vi)
    v1 = conv1_w.reshape(-1)
    w1pc = jnp.zeros((S * F, 700), jnp.float32).at[
        np.array(r_pc), np.array(c_pc)].set(v1[np.array(v_pc)])
    w1ad = jnp.zeros((S * F, 700), jnp.float32).at[
        np.array(r_ad), np.array(c_ad)].set(v1[np.array(v_ad)])

    rows, cols, vidx = [], [], []
    for o2 in range(10):
        for o in range(20):
            for j2 in range(17):
                for k in range(3):
                    rows.append(o * 35 + 2 * j2 + k)
                    cols.append(o2 * 17 + j2)
                    vidx.append((o2 * 20 + o) * 3 + k)
    w2m = jnp.zeros((700, 170), jnp.float32).at[
        np.array(rows), np.array(cols)].set(conv2_w.reshape(-1)[np.array(vidx)])

    rows, cols, vidx = [], [], []
    for o3 in range(6):
        for o2 in range(10):
            for j3 in range(8):
                for k in range(3):
                    rows.append(o2 * 17 + 2 * j3 + k)
                    cols.append((j3 % 2) * 24 + o3 * 4 + j3 // 2)
                    vidx.append((o3 * 10 + o2) * 3 + k)
    w3m = jnp.zeros((170, 48), jnp.float32).at[
        np.array(rows), np.array(cols)].set(conv3_w.reshape(-1)[np.array(vidx)])
    return w1pc, w1ad, w2m, w3m


def kernel(inp, pc_emb, addr_emb, enc_W, enc_b, conv1_w, conv1_b, conv2_w,
           conv2_b, conv3_w, conv3_b, dec_W, dec_b, rf1_W, rf1_b, rf2_W,
           rf2_b):
    f32 = jnp.float32
    enc_b2 = enc_b.reshape(1, EMB)

    p_all, p_ad = pl.pallas_call(
        _pk_body,
        out_shape=[
            jax.ShapeDtypeStruct((8, 256, EMB), f32),
            jax.ShapeDtypeStruct((4, 256, EMB), f32),
        ],
    )(pc_emb, addr_emb, enc_W, enc_b2)
    pspec = pl.BlockSpec((8, 256, EMB), lambda i, j: (0, 0, 0))
    t01 = pl.pallas_call(
        _t01_body,
        grid=(2, 4),
        in_specs=[pspec],
        out_specs=pl.BlockSpec((8192, F), lambda i, j: (i * 4 + j, 0)),
        out_shape=jax.ShapeDtypeStruct((65536, F), f32),
    )(p_all)
    t23 = pl.pallas_call(
        _t23_body,
        grid=(2, 8),
        in_specs=[pspec],
        out_specs=pl.BlockSpec((8192, F), lambda i, j: (i * 8 + j, 0)),
        out_shape=jax.ShapeDtypeStruct((131072, F), f32),
    )(p_all)

    vals = jnp.concatenate([inp[:, :, 0].reshape(-1), inp[:, :, 1].reshape(-1)])
    mesh = plsc.VectorSubcoreMesh(core_axis_name="c", subcore_axis_name="s")
    sc = pl.kernel(
        _sc_body,
        out_type=[jax.ShapeDtypeStruct((NTOT, F), f32)] * 2,
        mesh=mesh,
        scratch_types=[
            pltpu.VMEM((PER_W,), jnp.int32),
            pltpu.VMEM((CHUNK,), jnp.int32),
            pltpu.VMEM((CHUNK,), jnp.int32),
            pltpu.VMEM((CHUNK,), jnp.int32),
            pltpu.VMEM((CHUNK, F), f32),
            pltpu.VMEM((CHUNK, F), f32),
            pltpu.SemaphoreType.DMA,
            pltpu.SemaphoreType.DMA,
        ],
        compiler_params=pltpu.CompilerParams(
            needs_layout_passes=False, use_tc_tiling_on_sc=False),
    )
    DIAG_XLA_GATHER = True
    if DIAG_XLA_GATHER:
        st_off = jnp.concatenate([jnp.zeros(B * S, jnp.int32),
                                  jnp.ones(B * S, jnp.int32)])
        g01 = jnp.take(t01, (vals >> 16) + st_off * 32768, axis=0)
        g23 = jnp.take(t23, (vals & 0xFFFF) + st_off * 65536, axis=0)
        o01 = g01.reshape(2, B, S, F).transpose(2, 0, 1, 3).reshape(S, 2 * B, F)
        o23 = g23.reshape(2, B, S, F).transpose(2, 0, 1, 3).reshape(S, 2 * B, F)
    else:
        o01, o23 = sc(vals, t01, t23)
        o01 = o01.reshape(S, 2 * B, F)
        o23 = o23.reshape(S, 2 * B, F)

    w1pc, w1ad, w2m, w3m = _conv_mats(conv1_w, conv2_w, conv3_w)
    b1w = jnp.repeat(conv1_b, 35).reshape(1, 700)
    b2w = jnp.repeat(conv2_b, 17).reshape(1, 170)
    b3w = jnp.tile(conv3_b.reshape(6, 1), (2, 4)).reshape(1, 48)
    decm = dec_W.transpose(0, 2, 1)
    rf1m = rf1_W.T
    rf2m = rf2_W.T

    bb = 256
    grid = (B // bb,)
    nb = B // bb

    def full(*shape):
        return pl.BlockSpec(shape, lambda i: (0,) * len(shape))

    in_specs = [
        pl.BlockSpec((S, bb, F), lambda i: (0, i, 0)),
        pl.BlockSpec((S, bb, F), lambda i: (0, i, 0)),
        pl.BlockSpec((S, bb, F), lambda i: (0, i + nb, 0)),
        pl.BlockSpec((S, bb, F), lambda i: (0, i + nb, 0)),
        full(S * F, 700), full(S * F, 700), full(1, 700),
        full(700, 170), full(1, 170),
        full(170, 48), full(1, 48),
        full(4, 24, OUT), full(4, OUT),
        full(4, 256, EMB), full(1, EMB),
        full(3 * EMB, 10), full(1, 10),
        full(10, 2), full(1, 2),
    ]
    out_specs = (
        [pl.BlockSpec((bb, OUT), lambda i: (i, 0))] * 8
        + [pl.BlockSpec((bb, 2), lambda i: (i, 0))]
    )
    out_shape = (
        [jax.ShapeDtypeStruct((B, OUT), f32)] * 8
        + [jax.ShapeDtypeStruct((B, 2), f32)]
    )

    outs = pl.pallas_call(
        functools.partial(_dense_body, bb),
        grid=grid,
        in_specs=in_specs,
        out_specs=out_specs,
        out_shape=out_shape,
    )(o01, o23, o01, o23, w1pc, w1ad, b1w, w2m, b2w, w3m, b3w,
      decm, dec_b, p_ad, enc_b2, rf1m, rf1_b.reshape(1, 10),
      rf2m, rf2_b.reshape(1, 2))

    p0, p1, p2, p3, l0, l1, l2, l3, fr = outs
    return (p0, p1, p2, p3, l0, l1, l2, l3, fr[:, 0], fr[:, 1])


# speedup vs baseline: 3.6823x; 3.6823x over previous
import functools
import numpy as np
import jax
import jax.numpy as jnp
from jax import lax
from jax.experimental import pallas as pl
from jax.experimental.pallas import tpu as pltpu
from jax.experimental.pallas import tpu_sc as plsc

B = 4096
S = 20
EMB = 36
F = 48
OUT = 256
NTOT = 2 * B * S
PER_W = NTOT // 32
CHUNK = 128


def _pad48(x):
    z = jnp.zeros((x.shape[0], F - EMB), jnp.float32)
    return jnp.concatenate([x, z], axis=1)


def _pk_body(pc_emb, addr_emb, enc_W, enc_b, p_all, p_ad):
    dn = (((1,), (1,)), ((), ()))
    for st, emb in enumerate((pc_emb[...], addr_emb[...])):
        for i in range(4):
            w_i = enc_W[:, EMB * i:EMB * (i + 1)]
            p = lax.dot_general(emb[i], w_i, dn,
                                preferred_element_type=jnp.float32)
            if st == 1:
                p_ad[i] = p
            if i == 0:
                p = p + enc_b[...]
            p_all[st * 4 + i] = p


def _t01_body(p_all, t01):
    st = pl.program_id(0)
    g = pl.program_id(1)
    a0 = _pad48(p_all[st * 4, pl.ds(g * 32, 32), :])
    a1 = _pad48(p_all[st * 4 + 1])
    t01[...] = jnp.reshape(a0[:, None, :] + a1[None, :, :], (8192, F))


def _t23_body(p_all, t23):
    st = pl.program_id(0)
    g = pl.program_id(1)
    a2 = _pad48(p_all[st * 4 + 2, pl.ds(g * 32, 32), :])
    a3 = _pad48(p_all[st * 4 + 3])
    t23[...] = jnp.reshape(a2[:, None, :] + a3[None, :, :], (8192, F))


def _sc_body(vals_hbm, t01_hbm, t23_hbm, o01_hbm, o23_hbm,
             vals_v, idx01_v, idx23_v, row_v, g01_v, g23_v, sem1, sem2):
    wid = lax.axis_index("s") * 2 + lax.axis_index("c")
    st = wid // 16
    base = wid * PER_W
    nloc0 = (wid % 16) * PER_W
    off01 = st * 32768
    off23 = st * 65536
    rowoff = st * B
    pltpu.sync_copy(vals_hbm.at[pl.ds(base, PER_W)], vals_v)

    @pl.loop(0, PER_W, step=CHUNK)
    def _chunk(co):
        @pl.loop(0, CHUNK, step=16)
        def _idx(k):
            v = vals_v[pl.ds(co + k, 16)]
            idx01_v[pl.ds(k, 16)] = (v >> 16) + off01
            idx23_v[pl.ds(k, 16)] = (v & 0xFFFF) + off23
            n = (nloc0 + co + k) + lax.iota(jnp.int32, 16)
            row_v[pl.ds(k, 16)] = (n % S) * (2 * B) + rowoff + (n // S)

        c1 = pltpu.async_copy(t01_hbm.at[idx01_v], g01_v, sem1)
        c2 = pltpu.async_copy(t23_hbm.at[idx23_v], g23_v, sem2)
        c1.wait()
        c2.wait()
        c3 = pltpu.async_copy(g01_v, o01_hbm.at[row_v], sem1)
        c4 = pltpu.async_copy(g23_v, o23_hbm.at[row_v], sem2)
        c3.wait()
        c4.wait()


def _dense_body(bb, o01pc, o23pc, o01ad, o23ad, w1pc, w1ad, b1w, w2m, b2w,
                w3m, b3w, decm, dec_b, p_ad, enc_b, rf1m, rf1b, rf2m, rf2b,
                probs0, probs1, probs2, probs3, log0, log1, log2, log3, fr):
    def widen(a, c):
        x = jax.nn.sigmoid(a[...] + c[...])
        pieces = [x[s] for s in range(S)]
        wide = jnp.concatenate(pieces, axis=1)
        d = pieces[0]
        for s in range(1, S):
            d = d + pieces[s]
        return wide, d * (1.0 / S)

    xp, dist_pc = widen(o01pc, o23pc)
    xa, dist_ad = widen(o01ad, o23ad)

    h1 = jnp.dot(xp, w1pc[...], preferred_element_type=jnp.float32) \
        + jnp.dot(xa, w1ad[...], preferred_element_type=jnp.float32) \
        + b1w[...]
    h2 = jnp.dot(h1, w2m[...], preferred_element_type=jnp.float32) + b2w[...]
    h3 = jnp.dot(h2, w3m[...], preferred_element_type=jnp.float32) + b3w[...]
    h24 = jnp.maximum(h3[:, 0:24], h3[:, 24:48])

    probs_refs = (probs0, probs1, probs2, probs3)
    log_refs = (log0, log1, log2, log3)
    fe_pre = None
    for i in range(4):
        logits = jnp.dot(h24, decm[i], preferred_element_type=jnp.float32) \
            + dec_b[i][None, :]
        log_refs[i][...] = logits
        ls = logits / 0.01
        m = jnp.max(ls, axis=1, keepdims=True)
        e = jnp.exp(ls - m)
        p = e / jnp.sum(e, axis=1, keepdims=True)
        probs_refs[i][...] = p
        contrib = jnp.dot(p, p_ad[i], preferred_element_type=jnp.float32)
        fe_pre = contrib if fe_pre is None else fe_pre + contrib

    fe = jax.nn.sigmoid(fe_pre + enc_b[...])
    feat = jnp.concatenate([fe, dist_pc[:, :EMB], dist_ad[:, :EMB]], axis=1)
    r1 = jnp.maximum(
        jnp.dot(feat, rf1m[...], preferred_element_type=jnp.float32)
        + rf1b[...], 0.0)
    fr[...] = jnp.dot(r1, rf2m[...], preferred_element_type=jnp.float32) \
        + rf2b[...]


def _conv_mats(conv1_w, conv2_w, conv3_w):
    def rep(a, r0, r1):
        return jnp.repeat(jnp.repeat(a, r0, axis=0), r1, axis=1)

    f_row = np.arange(S * F) % F
    j_col = np.arange(700) % 35
    def w1_half(off):
        acc = None
        for k in range(3):
            m = ((f_row[:, None] + off == 2 * j_col[None, :] + k)
                 & (f_row[:, None] < EMB)).astype(np.float32)
            t = jnp.asarray(m) * rep(conv1_w[:, :, k].T, F, 35)
            acc = t if acc is None else acc + t
        return acc
    w1pc = w1_half(0)
    w1ad = w1_half(EMB)

    j_row2 = np.arange(700) % 35
    j2_col = np.arange(170) % 17
    acc = None
    for k in range(3):
        m = (j_row2[:, None] == 2 * j2_col[None, :] + k).astype(np.float32)
        t = jnp.asarray(m) * rep(conv2_w[:, :, k].T, 35, 17)
        acc = t if acc is None else acc + t
    w2m = acc

    j2_row = np.arange(170) % 17
    c3 = np.arange(48)
    j3_col = 2 * (c3 % 4) + c3 // 24
    acc = None
    for k in range(3):
        m = (j2_row[:, None] == 2 * j3_col[None, :] + k).astype(np.float32)
        v = jnp.tile(jnp.repeat(conv3_w[:, :, k].T, 4, axis=1), (1, 2))
        t = jnp.asarray(m) * jnp.repeat(v, 17, axis=0)
        acc = t if acc is None else acc + t
    w3m = acc
    return w1pc, w1ad, w2m, w3m


def kernel(inp, pc_emb, addr_emb, enc_W, enc_b, conv1_w, conv1_b, conv2_w,
           conv2_b, conv3_w, conv3_b, dec_W, dec_b, rf1_W, rf1_b, rf2_W,
           rf2_b):
    f32 = jnp.float32
    enc_b2 = enc_b.reshape(1, EMB)

    p_all, p_ad = pl.pallas_call(
        _pk_body,
        out_shape=[
            jax.ShapeDtypeStruct((8, 256, EMB), f32),
            jax.ShapeDtypeStruct((4, 256, EMB), f32),
        ],
    )(pc_emb, addr_emb, enc_W, enc_b2)
    pspec = pl.BlockSpec((8, 256, EMB), lambda i, j: (0, 0, 0))
    t01 = pl.pallas_call(
        _t01_body,
        grid=(2, 4),
        in_specs=[pspec],
        out_specs=pl.BlockSpec((8192, F), lambda i, j: (i * 4 + j, 0)),
        out_shape=jax.ShapeDtypeStruct((65536, F), f32),
    )(p_all)
    t23 = pl.pallas_call(
        _t23_body,
        grid=(2, 8),
        in_specs=[pspec],
        out_specs=pl.BlockSpec((8192, F), lambda i, j: (i * 8 + j, 0)),
        out_shape=jax.ShapeDtypeStruct((131072, F), f32),
    )(p_all)

    vals = jnp.concatenate([inp[:, :, 0].reshape(-1), inp[:, :, 1].reshape(-1)])
    mesh = plsc.VectorSubcoreMesh(core_axis_name="c", subcore_axis_name="s")
    sc = pl.kernel(
        _sc_body,
        out_type=[jax.ShapeDtypeStruct((NTOT, F), f32)] * 2,
        mesh=mesh,
        scratch_types=[
            pltpu.VMEM((PER_W,), jnp.int32),
            pltpu.VMEM((CHUNK,), jnp.int32),
            pltpu.VMEM((CHUNK,), jnp.int32),
            pltpu.VMEM((CHUNK,), jnp.int32),
            pltpu.VMEM((CHUNK, F), f32),
            pltpu.VMEM((CHUNK, F), f32),
            pltpu.SemaphoreType.DMA,
            pltpu.SemaphoreType.DMA,
        ],
        compiler_params=pltpu.CompilerParams(
            needs_layout_passes=False, use_tc_tiling_on_sc=False),
    )
    o01, o23 = sc(vals, t01, t23)
    o01 = o01.reshape(S, 2 * B, F)
    o23 = o23.reshape(S, 2 * B, F)

    w1pc, w1ad, w2m, w3m = _conv_mats(conv1_w, conv2_w, conv3_w)
    b1w = jnp.repeat(conv1_b, 35).reshape(1, 700)
    b2w = jnp.repeat(conv2_b, 17).reshape(1, 170)
    b3w = jnp.tile(conv3_b.reshape(6, 1), (2, 4)).reshape(1, 48)
    decm = dec_W.transpose(0, 2, 1)
    rf1m = rf1_W.T
    rf2m = rf2_W.T

    bb = 256
    grid = (B // bb,)
    nb = B // bb

    def full(*shape):
        return pl.BlockSpec(shape, lambda i: (0,) * len(shape))

    in_specs = [
        pl.BlockSpec((S, bb, F), lambda i: (0, i, 0)),
        pl.BlockSpec((S, bb, F), lambda i: (0, i, 0)),
        pl.BlockSpec((S, bb, F), lambda i: (0, i + nb, 0)),
        pl.BlockSpec((S, bb, F), lambda i: (0, i + nb, 0)),
        full(S * F, 700), full(S * F, 700), full(1, 700),
        full(700, 170), full(1, 170),
        full(170, 48), full(1, 48),
        full(4, 24, OUT), full(4, OUT),
        full(4, 256, EMB), full(1, EMB),
        full(3 * EMB, 10), full(1, 10),
        full(10, 2), full(1, 2),
    ]
    out_specs = (
        [pl.BlockSpec((bb, OUT), lambda i: (i, 0))] * 8
        + [pl.BlockSpec((bb, 2), lambda i: (i, 0))]
    )
    out_shape = (
        [jax.ShapeDtypeStruct((B, OUT), f32)] * 8
        + [jax.ShapeDtypeStruct((B, 2), f32)]
    )

    outs = pl.pallas_call(
        functools.partial(_dense_body, bb),
        grid=grid,
        in_specs=in_specs,
        out_specs=out_specs,
        out_shape=out_shape,
    )(o01, o23, o01, o23, w1pc, w1ad, b1w, w2m, b2w, w3m, b3w,
      decm, dec_b, p_ad, enc_b2, rf1m, rf1_b.reshape(1, 10),
      rf2m, rf2_b.reshape(1, 2))

    p0, p1, p2, p3, l0, l1, l2, l3, fr = outs
    return (p0, p1, p2, p3, l0, l1, l2, l3, fr[:, 0], fr[:, 1])
